# baseline (speedup 1.0000x reference)
import functools
import math

import jax
import jax.numpy as jnp
from jax import lax
from jax.experimental import pallas as pl
from jax.experimental.pallas import tpu as pltpu
from jax.experimental.pallas import tpu_sc as plsc

_NC = 2
_NS = 16
_NW = _NC * _NS
_L = 16

_SUB = 8
_LANE = 128
_CBLK = 2
_BU = _CBLK * _LANE


def _embed_lookup(idx_hm, table, h_dim, b_dim):
    n, = idx_hm.shape
    v_dim, d = table.shape
    assert d % _SUB == 0
    d8 = d // _SUB
    cb_per_h = b_dim // _BU
    n_units = h_dim * cb_per_h
    upw = n_units // _NW
    assert upw % 2 == 0
    scale = math.sqrt(d)
    h_words = d8 * (b_dim // _LANE) * _SUB * _LANE

    mesh = plsc.VectorSubcoreMesh(core_axis_name="c", subcore_axis_name="s")

    @functools.partial(
        pl.kernel,
        mesh=mesh,
        out_type=jax.ShapeDtypeStruct((h_dim, d8, b_dim // _LANE, _SUB, _LANE),
                                      jnp.float32),
        scratch_types=[
            [pltpu.VMEM((_BU,), jnp.int32) for _ in range(2)],
            [pltpu.VMEM((_BU, d), jnp.float32) for _ in range(2)],
            [pltpu.VMEM((d8, _CBLK, _SUB, _LANE), jnp.float32)
             for _ in range(2)],
            [pltpu.SemaphoreType.DMA for _ in range(2)],
            [pltpu.SemaphoreType.DMA for _ in range(2)],
            [pltpu.SemaphoreType.DMA for _ in range(2)],
        ],
        compiler_params=pltpu.CompilerParams(use_tc_tiling_on_sc=False,
                                             needs_layout_passes=False),
    )
    def run(idx_hbm, table_hbm, out_hbm, idx_v, rows_v, out_v,
            gsem, isem, osem):
        wid = lax.axis_index("s") * _NC + lax.axis_index("c")
        u0 = wid * upw
        iota = jnp.arange(_L, dtype=jnp.int32)

        def fire_gather(b, u):
            pltpu.async_copy(table_hbm.at[idx_v[b]], rows_v[b], gsem[b])

        def fire_idx(b, u):
            pltpu.async_copy(idx_hbm.at[pl.ds(u * _BU, _BU)], idx_v[b],
                             isem[b])

        def body(b, u, drain, refill):
            h = u // cb_per_h
            cb = u % cb_per_h
            pltpu.make_async_copy(table_hbm.at[idx_v[b]], rows_v[b],
                                  gsem[b]).wait()
            if refill:
                fire_idx(b, u + 2)
            if drain:
                pltpu.make_async_copy(
                    out_v[b], out_hbm.at[0, :, pl.ds(0, _CBLK), :, :],
                    osem[b]).wait()

            @pl.loop(0, d8 * _CBLK * _SUB * (_LANE // _L), unroll=8)
            def _tr(t):
                k8 = t // (_CBLK * _SUB * 8)
                c = (t // (_SUB * 8)) % _CBLK
                s = (t // 8) % _SUB
                lv = t % 8
                rows = c * _LANE + lv * _L + iota
                cols = jnp.full((_L,), k8 * _SUB + s, dtype=jnp.int32)
                vec = plsc.load_gather(rows_v[b], [rows, cols])
                out_v[b][k8, c, s, pl.ds(lv * _L, _L)] = vec * scale

            pltpu.async_copy(out_v[b],
                             out_hbm.at[h, :, pl.ds(cb * _CBLK, _CBLK), :, :],
                             osem[b])
            if refill:
                pltpu.make_async_copy(idx_hbm.at[pl.ds(0, _BU)], idx_v[b],
                                      isem[b]).wait()
                fire_gather(b, u + 2)

        for b in range(2):
            pltpu.sync_copy(idx_hbm.at[pl.ds((u0 + b) * _BU, _BU)], idx_v[b])
            fire_gather(b, u0 + b)
        for b in range(2):
            body(b, u0 + b, drain=False, refill=True)

        @pl.loop(2, upw - 2, step=2)
        def _main(i):
            for b in range(2):
                body(b, u0 + i + b, drain=True, refill=True)

        for b in range(2):
            body(b, u0 + upw - 2 + b, drain=True, refill=False)
        for b in range(2):
            pltpu.make_async_copy(out_v[b],
                                  out_hbm.at[0, :, pl.ds(0, _CBLK), :, :],
                                  osem[b]).wait()

    return run(idx_hm, table)


def kernel(x, table):
    b_dim, h_dim = x.shape
    v_dim, d = table.shape
    idx_hm = jnp.transpose(x).astype(jnp.int32).reshape(b_dim * h_dim)
    out5 = _embed_lookup(idx_hm, table, h_dim, b_dim)
    return (out5.transpose(2, 4, 0, 1, 3)
            .reshape(b_dim, h_dim, d))


# speedup vs baseline: 1.1264x; 1.1264x over previous
import functools
import math

import jax
import jax.numpy as jnp
from jax import lax
from jax.experimental import pallas as pl
from jax.experimental.pallas import tpu as pltpu
from jax.experimental.pallas import tpu_sc as plsc

_NC = 2
_NS = 16
_NW = _NC * _NS
_L = 16

_SUB = 8
_LANE = 128
_CBLK = 2
_BU = _CBLK * _LANE


def _embed_lookup(idx_hm, table, h_dim, b_dim):
    v_dim, d = table.shape
    assert d % _L == 0
    d8 = d // _SUB
    cb_per_h = b_dim // _BU
    n_units = h_dim * cb_per_h
    upw = n_units // _NW
    assert upw % 2 == 0
    scale = math.sqrt(d)
    uw = d8 * _CBLK * _SUB * _LANE
    tw = _CBLK * _SUB * _LANE
    h_words = d8 * (b_dim // _LANE) * _SUB * _LANE
    n_vregs = _BU * d // _L

    mesh = plsc.VectorSubcoreMesh(core_axis_name="c", subcore_axis_name="s")

    @functools.partial(
        pl.kernel,
        mesh=mesh,
        out_type=jax.ShapeDtypeStruct((h_dim * h_words,), jnp.float32),
        scratch_types=[
            [pltpu.VMEM((_BU,), jnp.int32) for _ in range(2)],
            [pltpu.VMEM((_BU, d), jnp.float32) for _ in range(2)],
            [pltpu.VMEM((uw,), jnp.float32) for _ in range(2)],
            [pltpu.SemaphoreType.DMA for _ in range(2)],
            [pltpu.SemaphoreType.DMA for _ in range(2)],
            [pltpu.SemaphoreType.DMA for _ in range(2)],
        ],
        compiler_params=pltpu.CompilerParams(use_tc_tiling_on_sc=False,
                                             needs_layout_passes=False),
    )
    def run(idx_hbm, table_hbm, out_hbm, idx_v, rows_v, out_v,
            gsem, isem, osem):
        wid = lax.axis_index("s") * _NC + lax.axis_index("c")
        u0 = wid * upw
        iota = jnp.arange(_L, dtype=jnp.int32)
        pattern = (iota // _SUB) * tw + (iota % _SUB) * _LANE

        def fire_gather(b, u):
            pltpu.async_copy(table_hbm.at[idx_v[b]], rows_v[b], gsem[b])

        def fire_idx(b, u):
            pltpu.async_copy(idx_hbm.at[pl.ds(u * _BU, _BU)], idx_v[b],
                             isem[b])

        def drain_out(b):
            pltpu.make_async_copy(out_v[b], out_hbm.at[pl.ds(0, uw)],
                                  osem[b]).wait()

        def body(b, u, drain, refill):
            h = u // cb_per_h
            cb = u % cb_per_h
            pltpu.make_async_copy(table_hbm.at[idx_v[b]], rows_v[b],
                                  gsem[b]).wait()
            if refill:
                fire_idx(b, u + 2)
            if drain:
                drain_out(b)

            @pl.loop(0, n_vregs, unroll=8)
            def _tr(t):
                r = t // (d // _L)
                k = t % (d // _L)
                base = (r & (_LANE - 1)) + (r // _LANE) * (_SUB * _LANE) \
                    + k * (_L // _SUB) * tw
                vec = rows_v[b][r, pl.ds(k * _L, _L)] * scale
                plsc.store_scatter(out_v[b], [pattern + base], vec)

            out0 = h * h_words + cb * tw
            for k8 in range(d8):
                pltpu.async_copy(
                    out_v[b].at[pl.ds(k8 * tw, tw)],
                    out_hbm.at[pl.ds(out0 + k8 * cb_per_h * tw, tw)],
                    osem[b])
            if refill:
                pltpu.make_async_copy(idx_hbm.at[pl.ds(0, _BU)], idx_v[b],
                                      isem[b]).wait()
                fire_gather(b, u + 2)

        for b in range(2):
            pltpu.sync_copy(idx_hbm.at[pl.ds((u0 + b) * _BU, _BU)], idx_v[b])
            fire_gather(b, u0 + b)
        for b in range(2):
            body(b, u0 + b, drain=False, refill=True)

        @pl.loop(2, upw - 2, step=2)
        def _main(i):
            for b in range(2):
                body(b, u0 + i + b, drain=True, refill=True)

        for b in range(2):
            body(b, u0 + upw - 2 + b, drain=True, refill=False)
        for b in range(2):
            drain_out(b)

    return run(idx_hm, table)


def kernel(x, table):
    b_dim, h_dim = x.shape
    v_dim, d = table.shape
    idx_hm = jnp.transpose(x).astype(jnp.int32).reshape(b_dim * h_dim)
    out = _embed_lookup(idx_hm, table, h_dim, b_dim)
    out5 = out.reshape(h_dim, d // _SUB, b_dim // _LANE, _SUB, _LANE)
    return (out5.transpose(2, 4, 0, 1, 3)
            .reshape(b_dim, h_dim, d))


# speedup vs baseline: 1.3627x; 1.2098x over previous
import functools
import math

import jax
import jax.numpy as jnp
from jax import lax
from jax.experimental import pallas as pl
from jax.experimental.pallas import tpu as pltpu
from jax.experimental.pallas import tpu_sc as plsc

_NC = 2
_NS = 16
_NW = _NC * _NS
_L = 16

_SUB = 8
_LANE = 128
_CBLK = 2
_BU = _CBLK * _LANE


def _embed_lookup(idx_hm, table, h_dim, b_dim):
    v_dim, d = table.shape
    assert d % _L == 0
    d8 = d // _SUB
    cb_per_h = b_dim // _BU
    n_units = h_dim * cb_per_h
    upw = n_units // _NW
    assert upw % 2 == 0
    scale = math.sqrt(d)
    uw = d8 * _CBLK * _SUB * _LANE
    tw = _CBLK * _SUB * _LANE
    h_words = d8 * (b_dim // _LANE) * _SUB * _LANE
    n_vregs = _BU * d // _L

    mesh = plsc.VectorSubcoreMesh(core_axis_name="c", subcore_axis_name="s")

    @functools.partial(
        pl.kernel,
        mesh=mesh,
        out_type=jax.ShapeDtypeStruct((h_dim * h_words,), jnp.float32),
        scratch_types=[
            [pltpu.VMEM((_BU,), jnp.int32) for _ in range(2)],
            [pltpu.VMEM((_BU, d), jnp.float32) for _ in range(2)],
            [pltpu.VMEM((uw,), jnp.float32) for _ in range(2)],
            [pltpu.SemaphoreType.DMA for _ in range(2)],
            [pltpu.SemaphoreType.DMA for _ in range(2)],
            [pltpu.SemaphoreType.DMA for _ in range(2)],
        ],
        compiler_params=pltpu.CompilerParams(use_tc_tiling_on_sc=False,
                                             needs_layout_passes=False),
    )
    def run(idx_hbm, table_hbm, out_hbm, idx_v, rows_v, out_v,
            gsem, isem, osem):
        wid = lax.axis_index("s") * _NC + lax.axis_index("c")
        u0 = wid * upw
        iota = jnp.arange(_L, dtype=jnp.int32)
        pattern = (iota // _SUB) * tw + (iota % _SUB) * _LANE

        def fire_gather(b, u):
            pltpu.async_copy(table_hbm.at[idx_v[b]], rows_v[b], gsem[b])

        def fire_idx(b, u):
            pltpu.async_copy(idx_hbm.at[pl.ds(u * _BU, _BU)], idx_v[b],
                             isem[b])

        def drain_out(b):
            pltpu.make_async_copy(out_v[b], out_hbm.at[pl.ds(0, uw)],
                                  osem[b]).wait()

        def body(b, u, drain, refill):
            h = u // cb_per_h
            cb = u % cb_per_h
            pltpu.make_async_copy(table_hbm.at[idx_v[b]], rows_v[b],
                                  gsem[b]).wait()
            if refill:
                fire_idx(b, u + 2)
            if drain:
                drain_out(b)

            @plsc.parallel_loop(0, n_vregs, unroll=8)
            def _tr(t):
                r = t // (d // _L)
                k = t % (d // _L)
                base = (r & (_LANE - 1)) + (r // _LANE) * (_SUB * _LANE) \
                    + k * (_L // _SUB) * tw
                vec = rows_v[b][r, pl.ds(k * _L, _L)] * scale
                plsc.store_scatter(out_v[b], [pattern + base], vec)

            out0 = h * h_words + cb * tw
            for k8 in range(d8):
                pltpu.async_copy(
                    out_v[b].at[pl.ds(k8 * tw, tw)],
                    out_hbm.at[pl.ds(out0 + k8 * cb_per_h * tw, tw)],
                    osem[b])
            if refill:
                pltpu.make_async_copy(idx_hbm.at[pl.ds(0, _BU)], idx_v[b],
                                      isem[b]).wait()
                fire_gather(b, u + 2)

        for b in range(2):
            pltpu.sync_copy(idx_hbm.at[pl.ds((u0 + b) * _BU, _BU)], idx_v[b])
            fire_gather(b, u0 + b)
        for b in range(2):
            body(b, u0 + b, drain=False, refill=True)

        @pl.loop(2, upw - 2, step=2)
        def _main(i):
            for b in range(2):
                body(b, u0 + i + b, drain=True, refill=True)

        for b in range(2):
            body(b, u0 + upw - 2 + b, drain=True, refill=False)
        for b in range(2):
            drain_out(b)

    return run(idx_hm, table)


def kernel(x, table):
    b_dim, h_dim = x.shape
    v_dim, d = table.shape
    idx_hm = jnp.transpose(x).astype(jnp.int32).reshape(b_dim * h_dim)
    out = _embed_lookup(idx_hm, table, h_dim, b_dim)
    out5 = out.reshape(h_dim, d // _SUB, b_dim // _LANE, _SUB, _LANE)
    return (out5.transpose(2, 4, 0, 1, 3)
            .reshape(b_dim, h_dim, d))


# speedup vs baseline: 2.5542x; 1.8744x over previous
import functools
import math

import jax
import jax.numpy as jnp
import numpy as np
from jax import lax
from jax.experimental import pallas as pl
from jax.experimental.pallas import tpu as pltpu
from jax.experimental.pallas import tpu_sc as plsc

_NC = 2
_NS = 16
_NW = _NC * _NS
_L = 16

_SUB = 8
_LANE = 128
_CBLK = 2
_BU = _CBLK * _LANE


def _embed_lookup(idx_hm, table, h_dim, b_dim):
    v_dim, d = table.shape
    assert d % _L == 0
    d8 = d // _SUB
    cb_per_h = b_dim // _BU
    n_units = h_dim * cb_per_h
    upw = n_units // _NW
    assert upw % 2 == 0
    scale = math.sqrt(d)
    uw = d8 * _CBLK * _SUB * _LANE
    tw = _CBLK * _SUB * _LANE
    h_words = d8 * (b_dim // _LANE) * _SUB * _LANE
    n_vregs = _BU * d // _L

    mesh = plsc.VectorSubcoreMesh(core_axis_name="c", subcore_axis_name="s")

    @functools.partial(
        pl.kernel,
        mesh=mesh,
        out_type=jax.ShapeDtypeStruct((h_dim * h_words,), jnp.float32),
        scratch_types=[
            [pltpu.VMEM((_BU,), jnp.int32) for _ in range(2)],
            [pltpu.VMEM((_BU, d), jnp.float32) for _ in range(2)],
            [pltpu.VMEM((uw,), jnp.float32) for _ in range(2)],
            [pltpu.SemaphoreType.DMA for _ in range(2)],
            [pltpu.SemaphoreType.DMA for _ in range(2)],
            [pltpu.SemaphoreType.DMA for _ in range(2)],
        ],
        compiler_params=pltpu.CompilerParams(use_tc_tiling_on_sc=False,
                                             needs_layout_passes=False),
    )
    def run(idx_hbm, table_hbm, out_hbm, idx_v, rows_v, out_v,
            gsem, isem, osem):
        wid = lax.axis_index("s") * _NC + lax.axis_index("c")
        u0 = wid * upw
        iota = jnp.arange(_L, dtype=jnp.int32)
        dmv = [(iota + m) & (_L - 1) for m in range(_L)]
        patj = [iota + (dm // _SUB) * tw + (dm % _SUB) * _LANE for dm in dmv]

        def fire_gather(b, u):
            pltpu.async_copy(table_hbm.at[idx_v[b]], rows_v[b], gsem[b])

        def fire_idx(b, u):
            pltpu.async_copy(idx_hbm.at[pl.ds(u * _BU, _BU)], idx_v[b],
                             isem[b])

        def drain_out(b):
            pltpu.make_async_copy(out_v[b], out_hbm.at[pl.ds(0, uw)],
                                  osem[b]).wait()

        def body(b, u, drain, refill):
            h = u // cb_per_h
            cb = u % cb_per_h
            pltpu.make_async_copy(table_hbm.at[idx_v[b]], rows_v[b],
                                  gsem[b]).wait()
            if refill:
                fire_idx(b, u + 2)
            if drain:
                drain_out(b)

            @plsc.parallel_loop(0, _CBLK * (_LANE // _L) * (d // _L),
                                unroll=1)
            def _tr(q):
                c = q // ((_LANE // _L) * (d // _L))
                lb = (q // (d // _L)) % (_LANE // _L)
                dd = q % (d // _L)
                rvec = iota + (c * _LANE + lb * _L)
                base_d = c * (_SUB * _LANE) + lb * _L + dd * (_L // _SUB) * tw
                for m in range(_L):
                    vec = plsc.load_gather(rows_v[b],
                                           [rvec, dmv[m] + dd * _L])
                    plsc.store_scatter(out_v[b], [patj[m] + base_d],
                                       vec * scale)

            out0 = h * h_words + cb * tw
            for k8 in range(d8):
                pltpu.async_copy(
                    out_v[b].at[pl.ds(k8 * tw, tw)],
                    out_hbm.at[pl.ds(out0 + k8 * cb_per_h * tw, tw)],
                    osem[b])
            if refill:
                pltpu.make_async_copy(idx_hbm.at[pl.ds(0, _BU)], idx_v[b],
                                      isem[b]).wait()
                fire_gather(b, u + 2)

        for b in range(2):
            pltpu.sync_copy(idx_hbm.at[pl.ds((u0 + b) * _BU, _BU)], idx_v[b])
            fire_gather(b, u0 + b)
        for b in range(2):
            body(b, u0 + b, drain=False, refill=True)

        @pl.loop(2, upw - 2, step=2)
        def _main(i):
            for b in range(2):
                body(b, u0 + i + b, drain=True, refill=True)

        for b in range(2):
            body(b, u0 + upw - 2 + b, drain=True, refill=False)
        for b in range(2):
            drain_out(b)

    return run(idx_hm, table)


def kernel(x, table):
    b_dim, h_dim = x.shape
    v_dim, d = table.shape
    idx_hm = jnp.transpose(x).astype(jnp.int32).reshape(b_dim * h_dim)
    out = _embed_lookup(idx_hm, table, h_dim, b_dim)
    out5 = out.reshape(h_dim, d // _SUB, b_dim // _LANE, _SUB, _LANE)
    return (out5.transpose(2, 4, 0, 1, 3)
            .reshape(b_dim, h_dim, d))


# speedup vs baseline: 2.5543x; 1.0001x over previous
import functools
import math

import jax
import jax.numpy as jnp
from jax import lax
from jax.experimental import pallas as pl
from jax.experimental.pallas import tpu as pltpu
from jax.experimental.pallas import tpu_sc as plsc

_NC = 2
_NS = 16
_NW = _NC * _NS
_L = 16

_SUB = 8
_LANE = 128
_CBLK = 2
_BU = _CBLK * _LANE


def _embed_lookup(idx_hm, table, h_dim, b_dim):
    v_dim, d = table.shape
    d8 = d // _SUB
    cb_per_h = b_dim // _BU
    n_units = h_dim * cb_per_h
    upw = n_units // _NW
    assert upw % 2 == 0
    scale = math.sqrt(d)
    uw = d8 * _CBLK * _SUB * _LANE
    tw = _CBLK * _SUB * _LANE
    h_words = d8 * (b_dim // _LANE) * _SUB * _LANE

    mesh = plsc.VectorSubcoreMesh(core_axis_name="c", subcore_axis_name="s")

    @functools.partial(
        pl.kernel,
        mesh=mesh,
        out_type=jax.ShapeDtypeStruct((h_dim * h_words,), jnp.float32),
        scratch_types=[
            [pltpu.VMEM((_BU,), jnp.int32) for _ in range(2)],
            [pltpu.VMEM((_BU, d), jnp.float32) for _ in range(2)],
            [pltpu.VMEM((uw,), jnp.float32) for _ in range(2)],
            [pltpu.SemaphoreType.DMA for _ in range(2)],
            [pltpu.SemaphoreType.DMA for _ in range(2)],
            [pltpu.SemaphoreType.DMA for _ in range(2)],
        ],
        compiler_params=pltpu.CompilerParams(use_tc_tiling_on_sc=False,
                                             needs_layout_passes=False),
    )
    def run(idx_hbm, table_hbm, out_hbm, idx_v, rows_v, out_v,
            gsem, isem, osem):
        wid = lax.axis_index("s") * _NC + lax.axis_index("c")
        u0 = wid * upw
        iota = jnp.arange(_L, dtype=jnp.int32)
        dmv = [(iota + m) & (_L - 1) for m in range(_L)]
        patj = [iota + (dm // _SUB) * tw + (dm % _SUB) * _LANE for dm in dmv]

        def fire_gather(b):
            pltpu.async_copy(table_hbm.at[idx_v[b]], rows_v[b], gsem[b])

        def fire_idx(b, u):
            pltpu.async_copy(idx_hbm.at[pl.ds(u * _BU, _BU)], idx_v[b],
                             isem[b])

        def drain_out(b):
            pltpu.make_async_copy(out_v[b], out_hbm.at[pl.ds(0, uw)],
                                  osem[b]).wait()

        def body(b, u, drain, refill):
            h = u // cb_per_h
            cb = u % cb_per_h
            pltpu.make_async_copy(table_hbm.at[idx_v[b]], rows_v[b],
                                  gsem[b]).wait()
            if refill:
                fire_idx(b, u + 2)
            if drain:
                drain_out(b)

            @plsc.parallel_loop(0, _CBLK * (_LANE // _L) * (d // _L),
                                unroll=1)
            def _tr(q):
                c = q // ((_LANE // _L) * (d // _L))
                lb = (q // (d // _L)) % (_LANE // _L)
                dd = q % (d // _L)
                rvec = iota + (c * _LANE + lb * _L)
                base_d = c * (_SUB * _LANE) + lb * _L + dd * (_L // _SUB) * tw
                for m in range(_L):
                    vec = plsc.load_gather(rows_v[b],
                                           [rvec, dmv[m] + dd * _L])
                    plsc.store_scatter(out_v[b], [patj[m] + base_d],
                                       vec * scale)

            out0 = h * h_words + cb * tw
            for k8 in range(d8):
                pltpu.async_copy(
                    out_v[b].at[pl.ds(k8 * tw, tw)],
                    out_hbm.at[pl.ds(out0 + k8 * cb_per_h * tw, tw)],
                    osem[b])
            if refill:
                pltpu.make_async_copy(idx_hbm.at[pl.ds(0, _BU)], idx_v[b],
                                      isem[b]).wait()
                fire_gather(b)

        for b in range(2):
            pltpu.sync_copy(idx_hbm.at[pl.ds((u0 + b) * _BU, _BU)], idx_v[b])
            fire_gather(b)
        for b in range(2):
            body(b, u0 + b, drain=False, refill=True)

        @pl.loop(2, upw - 2, step=2)
        def _main(i):
            for b in range(2):
                body(b, u0 + i + b, drain=True, refill=True)

        for b in range(2):
            body(b, u0 + upw - 2 + b, drain=True, refill=False)
        for b in range(2):
            drain_out(b)

    return run(idx_hm, table)


def kernel(x, table):
    b_dim, h_dim = x.shape
    v_dim, d = table.shape
    idx_hm = jnp.transpose(x).astype(jnp.int32).reshape(b_dim * h_dim)
    out = _embed_lookup(idx_hm, table, h_dim, b_dim)
    out5 = out.reshape(h_dim, d // _SUB, b_dim // _LANE, _SUB, _LANE)
    return (out5.transpose(2, 4, 0, 1, 3)
            .reshape(b_dim, h_dim, d))
